# baseline (speedup 1.0000x reference)
import functools

import jax
import jax.numpy as jnp
from jax import lax
from jax.experimental import pallas as pl
from jax.experimental.pallas import tpu as pltpu
from jax.experimental.pallas import tpu_sc as plsc

NEG_SLOPE = 0.01

NC = 2
NS = 16
NW = NC * NS
CH = 128
HW = 16

_mesh = functools.partial(
    plsc.VectorSubcoreMesh, core_axis_name="c", subcore_axis_name="s",
    num_cores=NC, num_subcores=NS)


def _hist_kernel(npad, e, k):
    @functools.partial(
        pl.kernel,
        mesh=_mesh(),
        out_type=jax.ShapeDtypeStruct((NC, npad), jnp.float32),
        scratch_types=[
            pltpu.VMEM((4, 2, CH), jnp.int32),
            pltpu.VMEM((CH,), jnp.float32),
            pltpu.VMEM_SHARED((npad,), jnp.float32),
            [pltpu.SemaphoreType.DMA] * 4,
            pltpu.SemaphoreType.DMA,
            pltpu.SemaphoreType.DMA,
        ],
    )
    def hist(ei_hbm, dum_hbm, zer_hbm, ones_hbm, out_hbm, idx_v, ones_v,
             hist_sh, isems, ssem, zsem):
        c = lax.axis_index("c")
        s = lax.axis_index("s")
        wid = s * NC + c
        kfull = e // (NW * CH)
        nt = (e // CH) % NW
        kw = kfull + jnp.where(wid < nt, 1, 0)
        e0 = NW * kfull * CH
        base = wid * (kfull * CH)
        rows = npad // NS

        pltpu.async_copy(zer_hbm.at[pl.ds(s * rows, rows)],
                         hist_sh.at[pl.ds(s * rows, rows)], zsem)
        pltpu.sync_copy(ones_hbm, ones_v)

        def idx_load_fast(j, slot):
            pltpu.async_copy(ei_hbm.at[pl.ds(0, 2), pl.ds(base + j * CH, CH)],
                             idx_v.at[slot], isems[slot])

        def idx_load_tail(j, slot):
            off = jnp.where(j < kfull, base + j * CH, e0 + wid * CH)

            @pl.when(j < kw)
            def _():
                pltpu.async_copy(ei_hbm.at[pl.ds(0, 2), pl.ds(off, CH)],
                                 idx_v.at[slot], isems[slot])

            @pl.when(j >= kw)
            def _():
                pltpu.async_copy(dum_hbm, idx_v.at[slot], isems[slot])

        def idx_wait(slot):
            pltpu.make_async_copy(dum_hbm, idx_v.at[slot], isems[slot]).wait()

        for i in range(4):
            idx_load_fast(i, i)
        pltpu.make_async_copy(zer_hbm.at[pl.ds(s * rows, rows)],
                              hist_sh.at[pl.ds(s * rows, rows)], zsem).wait()
        plsc.subcore_barrier()

        def step(m, load):
            for i in range(4):
                idx_wait(i)
                pltpu.async_copy(ones_v, hist_sh.at[idx_v.at[i, 1]], ssem,
                                 add=True)
            for i in range(4):
                pltpu.make_async_copy(ones_v, hist_sh.at[idx_v.at[i, 1]],
                                      ssem).wait()
                if load is not None:
                    load(4 * m + i + 4, i)

        nm = k // 4
        lax.fori_loop(
            0, nm - 2,
            lambda m, cr: (step(m, idx_load_fast), cr)[1], 0, unroll=False)
        step(nm - 2, idx_load_tail)
        step(nm - 1, None)
        plsc.subcore_barrier()
        pltpu.sync_copy(hist_sh.at[pl.ds(s * rows, rows)],
                        out_hbm.at[c, pl.ds(s * rows, rows)])

    return hist


def _scatter_kernel(n, npad, d, e, k):
    assert k % 2 == 0

    assert k % 4 == 0
    nm = k // 4

    @functools.partial(
        pl.kernel,
        mesh=_mesh(),
        out_type=jax.ShapeDtypeStruct((NC, npad, d), jnp.float32),
        scratch_types=[
            pltpu.VMEM((4, 2, CH), jnp.int32),
            pltpu.VMEM((2, CH, d), jnp.float32),
            pltpu.VMEM_SHARED((npad, d), jnp.float32),
            [pltpu.SemaphoreType.DMA] * 4,
            [pltpu.SemaphoreType.DMA] * 2,
            [pltpu.SemaphoreType.DMA] * 2,
            pltpu.SemaphoreType.DMA,
        ],
    )
    def scat(y_hbm, ei_hbm, dum_hbm, zer_hbm, out_hbm,
             idx_v, buf_v, acc_sh, isems, gsems, ssems, zsem):
        c = lax.axis_index("c")
        s = lax.axis_index("s")
        wid = s * NC + c
        kfull = e // (NW * CH)
        nt = (e // CH) % NW
        kw = kfull + jnp.where(wid < nt, 1, 0)
        e0 = NW * kfull * CH
        base = wid * (kfull * CH)
        rows = npad // NS

        pltpu.async_copy(zer_hbm.at[pl.ds(s * rows, rows)],
                         acc_sh.at[pl.ds(s * rows, rows)], zsem)

        def idx_load_fast(j, slot):
            pltpu.async_copy(ei_hbm.at[pl.ds(0, 2), pl.ds(base + j * CH, CH)],
                             idx_v.at[slot], isems[slot])

        def idx_load_tail(j, slot):
            off = jnp.where(j < kfull, base + j * CH, e0 + wid * CH)

            @pl.when(j < kw)
            def _():
                pltpu.async_copy(ei_hbm.at[pl.ds(0, 2), pl.ds(off, CH)],
                                 idx_v.at[slot], isems[slot])

            @pl.when(j >= kw)
            def _():
                pltpu.async_copy(dum_hbm, idx_v.at[slot], isems[slot])

        def idx_wait(slot):
            pltpu.make_async_copy(dum_hbm, idx_v.at[slot], isems[slot]).wait()

        def gat_start(islot, bslot):
            pltpu.async_copy(y_hbm.at[idx_v.at[islot, 0]], buf_v.at[bslot],
                             gsems[bslot])

        def gat_wait(islot, bslot):
            pltpu.make_async_copy(y_hbm.at[idx_v.at[islot, 0]],
                                  buf_v.at[bslot], gsems[bslot]).wait()

        for i in range(4):
            idx_load_fast(i, i)
        idx_wait(0)
        gat_start(0, 0)
        pltpu.make_async_copy(zer_hbm.at[pl.ds(s * rows, rows)],
                              acc_sh.at[pl.ds(s * rows, rows)], zsem).wait()
        plsc.subcore_barrier()

        def scat_add(i):
            pltpu.async_copy(buf_v.at[i % 2], acc_sh.at[idx_v.at[i, 1]],
                             ssems[i % 2], add=True)

        def scat_wait(bslot, islot):
            pltpu.make_async_copy(buf_v.at[bslot],
                                  acc_sh.at[idx_v.at[islot, 1]],
                                  ssems[bslot]).wait()

        def step(m, loadfn, first=False, last=False):
            for i in range(4):
                gat_wait(i % 4, i % 2)
                scat_add(i)
                if not (last and i == 3):
                    idx_wait((i + 1) % 4)
                    if not (first and i == 0):
                        scat_wait((i + 1) % 2, (i + 3) % 4)
                        if loadfn is not None:
                            loadfn(4 * m + i + 3, (i + 3) % 4)
                    gat_start((i + 1) % 4, (i + 1) % 2)

        def load_static(c, slot):
            if c >= k:
                return
            if c < kfull:
                idx_load_fast(c, slot)
            else:
                idx_load_tail(c, slot)

        nm = k // 4
        step(0, idx_load_fast, first=True)
        lax.fori_loop(
            1, nm - 2,
            lambda m, cr: (step(m, idx_load_fast), cr)[1], 0, unroll=False)
        step(nm - 2, load_static)
        step(nm - 1, load_static, last=True)
        scat_wait(0, 2)
        scat_wait(1, 3)
        plsc.subcore_barrier()
        pltpu.sync_copy(acc_sh.at[pl.ds(s * rows, rows)],
                        out_hbm.at[c, pl.ds(s * rows, rows)])

    return scat


def _dis(h_ref, n):
    h = h_ref[...]
    deg = 1.0 + h[0, :n] + h[1, :n]
    return lax.rsqrt(deg)[:, None]


def _tc1_body(n, x_ref, w_ref, h_ref, y_ref):
    xw = jnp.dot(x_ref[...], w_ref[...], preferred_element_type=jnp.float32)
    y_ref[...] = xw * _dis(h_ref, n)


def _tc2_body(n, acc_ref, y1_ref, h_ref, w2_ref, b1_ref, y2_ref):
    dis = _dis(h_ref, n)
    o1 = dis * (acc_ref[0, :n, :] + acc_ref[1, :n, :] + y1_ref[...])
    o1 = o1 + b1_ref[...][None, :]
    hmat = jnp.where(o1 >= 0, o1, NEG_SLOPE * o1)
    y2_ref[...] = jnp.dot(hmat, w2_ref[...],
                          preferred_element_type=jnp.float32) * dis


def _tc3_body(n, acc_ref, y2_ref, h_ref, b2_ref, out_ref):
    dis = _dis(h_ref, n)
    o2 = dis * (acc_ref[0, :n, :] + acc_ref[1, :n, :] + y2_ref[...])
    out_ref[...] = o2 + b2_ref[...][None, :]


def kernel(x, edge_index, W1, b1, W2, b2):
    n, d = x.shape
    e = edge_index.shape[1]
    assert e % CH == 0
    npad = (n // (16 * NS) + 1) * 16 * NS
    ndump = npad - n
    k = -(-e // (NW * CH))
    k += (-k) % 4

    i128 = jnp.arange(CH, dtype=jnp.int32)
    dum = jnp.stack([(i128 * 79) % n, n + i128 % ndump])
    zer = jnp.zeros((npad, d), jnp.float32)
    zerh = jnp.zeros((npad,), jnp.float32)

    onesh = jnp.ones((CH,), jnp.float32)
    histp = _hist_kernel(npad, e, k)(edge_index, dum, zerh, onesh)
    scat = _scatter_kernel(n, npad, d, e, k)

    y1 = pl.pallas_call(
        functools.partial(_tc1_body, n),
        out_shape=jax.ShapeDtypeStruct((n, d), jnp.float32),
    )(x, W1, histp)

    acc1 = scat(y1, edge_index, dum, zer)

    y2 = pl.pallas_call(
        functools.partial(_tc2_body, n),
        out_shape=jax.ShapeDtypeStruct((n, d), jnp.float32),
    )(acc1, y1, histp, W2, b1)

    acc2 = scat(y2, edge_index, dum, zer)

    out = pl.pallas_call(
        functools.partial(_tc3_body, n),
        out_shape=jax.ShapeDtypeStruct((n, d), jnp.float32),
    )(acc2, y2, histp, b2)

    return out


# speedup vs baseline: 1.1539x; 1.1539x over previous
import functools

import jax
import jax.numpy as jnp
from jax import lax
from jax.experimental import pallas as pl
from jax.experimental.pallas import tpu as pltpu
from jax.experimental.pallas import tpu_sc as plsc

NEG_SLOPE = 0.01

NC = 2
NS = 16
NW = NC * NS
CH = 128
HW = 16

_mesh = functools.partial(
    plsc.VectorSubcoreMesh, core_axis_name="c", subcore_axis_name="s",
    num_cores=NC, num_subcores=NS)


def _hist_kernel(npad, e, k):
    @functools.partial(
        pl.kernel,
        mesh=_mesh(),
        out_type=jax.ShapeDtypeStruct((NC, npad), jnp.float32),
        scratch_types=[
            pltpu.VMEM((4, 2, CH), jnp.int32),
            pltpu.VMEM((CH,), jnp.float32),
            pltpu.VMEM_SHARED((npad,), jnp.float32),
            [pltpu.SemaphoreType.DMA] * 4,
            pltpu.SemaphoreType.DMA,
            pltpu.SemaphoreType.DMA,
        ],
    )
    def hist(ei_hbm, dum_hbm, zer_hbm, ones_hbm, out_hbm, idx_v, ones_v,
             hist_sh, isems, ssem, zsem):
        c = lax.axis_index("c")
        s = lax.axis_index("s")
        wid = s * NC + c
        kfull = e // (NW * CH)
        nt = (e // CH) % NW
        kw = kfull + jnp.where(wid < nt, 1, 0)
        e0 = NW * kfull * CH
        base = wid * (kfull * CH)
        rows = npad // NS

        pltpu.async_copy(zer_hbm.at[pl.ds(s * rows, rows)],
                         hist_sh.at[pl.ds(s * rows, rows)], zsem)
        pltpu.sync_copy(ones_hbm, ones_v)

        def idx_load_fast(j, slot):
            pltpu.async_copy(ei_hbm.at[pl.ds(0, 2), pl.ds(base + j * CH, CH)],
                             idx_v.at[slot], isems[slot])

        def idx_load_tail(j, slot):
            off = jnp.where(j < kfull, base + j * CH, e0 + wid * CH)

            @pl.when(j < kw)
            def _():
                pltpu.async_copy(ei_hbm.at[pl.ds(0, 2), pl.ds(off, CH)],
                                 idx_v.at[slot], isems[slot])

            @pl.when(j >= kw)
            def _():
                pltpu.async_copy(dum_hbm, idx_v.at[slot], isems[slot])

        def idx_wait(slot):
            pltpu.make_async_copy(dum_hbm, idx_v.at[slot], isems[slot]).wait()

        for i in range(4):
            idx_load_fast(i, i)
        pltpu.make_async_copy(zer_hbm.at[pl.ds(s * rows, rows)],
                              hist_sh.at[pl.ds(s * rows, rows)], zsem).wait()
        plsc.subcore_barrier()

        def step(m, load):
            for i in range(4):
                idx_wait(i)
                pltpu.async_copy(ones_v, hist_sh.at[idx_v.at[i, 1]], ssem,
                                 add=True)
            for i in range(4):
                pltpu.make_async_copy(ones_v, hist_sh.at[idx_v.at[i, 1]],
                                      ssem).wait()
                if load is not None:
                    load(4 * m + i + 4, i)

        nm = k // 4
        lax.fori_loop(
            0, nm - 2,
            lambda m, cr: (step(m, idx_load_fast), cr)[1], 0, unroll=False)
        step(nm - 2, idx_load_tail)
        step(nm - 1, None)
        plsc.subcore_barrier()
        pltpu.sync_copy(hist_sh.at[pl.ds(s * rows, rows)],
                        out_hbm.at[c, pl.ds(s * rows, rows)])

    return hist


def _scatter_kernel(n, npad, d, e, k):
    assert k % 2 == 0

    assert k % 4 == 0
    nm = k // 4

    @functools.partial(
        pl.kernel,
        mesh=_mesh(),
        out_type=jax.ShapeDtypeStruct((NC, npad, d), jnp.float32),
        scratch_types=[
            pltpu.VMEM((4, 2, CH), jnp.int32),
            pltpu.VMEM((2, CH, d), jnp.float32),
            pltpu.VMEM_SHARED((npad, d), jnp.float32),
            [pltpu.SemaphoreType.DMA] * 4,
            [pltpu.SemaphoreType.DMA] * 2,
            [pltpu.SemaphoreType.DMA] * 2,
            pltpu.SemaphoreType.DMA,
        ],
    )
    def scat(y_hbm, ei_hbm, dum_hbm, zer_hbm, out_hbm,
             idx_v, buf_v, acc_sh, isems, gsems, ssems, zsem):
        c = lax.axis_index("c")
        s = lax.axis_index("s")
        wid = s * NC + c
        kfull = e // (NW * CH)
        nt = (e // CH) % NW
        kw = kfull + jnp.where(wid < nt, 1, 0)
        e0 = NW * kfull * CH
        base = wid * (kfull * CH)
        rows = npad // NS

        pltpu.async_copy(zer_hbm.at[pl.ds(s * rows, rows)],
                         acc_sh.at[pl.ds(s * rows, rows)], zsem)

        def idx_load_fast(j, slot):
            pltpu.async_copy(ei_hbm.at[pl.ds(0, 2), pl.ds(base + j * CH, CH)],
                             idx_v.at[slot], isems[slot])

        def idx_load_tail(j, slot):
            off = jnp.where(j < kfull, base + j * CH, e0 + wid * CH)

            @pl.when(j < kw)
            def _():
                pltpu.async_copy(ei_hbm.at[pl.ds(0, 2), pl.ds(off, CH)],
                                 idx_v.at[slot], isems[slot])

            @pl.when(j >= kw)
            def _():
                pltpu.async_copy(dum_hbm, idx_v.at[slot], isems[slot])

        def idx_wait(slot):
            pltpu.make_async_copy(dum_hbm, idx_v.at[slot], isems[slot]).wait()

        def gat_start(islot, bslot):
            pltpu.async_copy(y_hbm.at[idx_v.at[islot, 0]], buf_v.at[bslot],
                             gsems[bslot])

        def gat_wait(islot, bslot):
            pltpu.make_async_copy(y_hbm.at[idx_v.at[islot, 0]],
                                  buf_v.at[bslot], gsems[bslot]).wait()

        for i in range(4):
            idx_load_fast(i, i)
        idx_wait(0)
        gat_start(0, 0)
        pltpu.make_async_copy(zer_hbm.at[pl.ds(s * rows, rows)],
                              acc_sh.at[pl.ds(s * rows, rows)], zsem).wait()
        plsc.subcore_barrier()

        def step(m, load, last=False):
            for i in range(4):
                if not (last and i == 3):
                    idx_wait((i + 1) % 4)
                    gat_start((i + 1) % 4, (i + 1) % 2)
                gat_wait(i % 4, i % 2)
                pltpu.sync_copy(buf_v.at[i % 2], acc_sh.at[idx_v.at[i, 1]],
                                add=True)
                if load is not None:
                    load(4 * m + i + 4, i)

        nm = k // 4
        lax.fori_loop(
            0, nm - 2,
            lambda m, cr: (step(m, idx_load_fast), cr)[1], 0, unroll=False)
        step(nm - 2, idx_load_tail)
        step(nm - 1, None, last=True)
        plsc.subcore_barrier()
        pltpu.sync_copy(acc_sh.at[pl.ds(s * rows, rows)],
                        out_hbm.at[c, pl.ds(s * rows, rows)])

    return scat


def _dis(h_ref, n):
    h = h_ref[...]
    deg = 1.0 + h[0, :n] + h[1, :n]
    return lax.rsqrt(deg)[:, None]


def _tc1_body(n, x_ref, w_ref, h_ref, y_ref):
    xw = jnp.dot(x_ref[...], w_ref[...], preferred_element_type=jnp.float32)
    y_ref[...] = xw * _dis(h_ref, n)


def _tc2_body(n, acc_ref, y1_ref, h_ref, w2_ref, b1_ref, y2_ref):
    dis = _dis(h_ref, n)
    o1 = dis * (acc_ref[0, :n, :] + acc_ref[1, :n, :] + y1_ref[...])
    o1 = o1 + b1_ref[...][None, :]
    hmat = jnp.where(o1 >= 0, o1, NEG_SLOPE * o1)
    y2_ref[...] = jnp.dot(hmat, w2_ref[...],
                          preferred_element_type=jnp.float32) * dis


def _tc3_body(n, acc_ref, y2_ref, h_ref, b2_ref, out_ref):
    dis = _dis(h_ref, n)
    o2 = dis * (acc_ref[0, :n, :] + acc_ref[1, :n, :] + y2_ref[...])
    out_ref[...] = o2 + b2_ref[...][None, :]


def kernel(x, edge_index, W1, b1, W2, b2):
    n, d = x.shape
    e = edge_index.shape[1]
    assert e % CH == 0
    npad = (n // (16 * NS) + 1) * 16 * NS
    ndump = npad - n
    k = -(-e // (NW * CH))
    k += (-k) % 4

    i128 = jnp.arange(CH, dtype=jnp.int32)
    dum = jnp.stack([(i128 * 79) % n, n + i128 % ndump])
    zer = jnp.zeros((npad, d), jnp.float32)
    zerh = jnp.zeros((npad,), jnp.float32)

    onesh = jnp.ones((CH,), jnp.float32)
    histp = _hist_kernel(npad, e, k)(edge_index, dum, zerh, onesh)
    scat = _scatter_kernel(n, npad, d, e, k)

    y1 = pl.pallas_call(
        functools.partial(_tc1_body, n),
        out_shape=jax.ShapeDtypeStruct((n, d), jnp.float32),
    )(x, W1, histp)

    acc1 = scat(y1, edge_index, dum, zer)

    y2 = pl.pallas_call(
        functools.partial(_tc2_body, n),
        out_shape=jax.ShapeDtypeStruct((n, d), jnp.float32),
    )(acc1, y1, histp, W2, b1)

    acc2 = scat(y2, edge_index, dum, zer)

    out = pl.pallas_call(
        functools.partial(_tc3_body, n),
        out_shape=jax.ShapeDtypeStruct((n, d), jnp.float32),
    )(acc2, y2, histp, b2)

    return out


# speedup vs baseline: 1.1800x; 1.0226x over previous
import functools

import jax
import jax.numpy as jnp
from jax import lax
from jax.experimental import pallas as pl
from jax.experimental.pallas import tpu as pltpu
from jax.experimental.pallas import tpu_sc as plsc

NEG_SLOPE = 0.01

NC = 2
NS = 16
NW = NC * NS
CH = 128
HW = 16

_mesh = functools.partial(
    plsc.VectorSubcoreMesh, core_axis_name="c", subcore_axis_name="s",
    num_cores=NC, num_subcores=NS)


def _hist_kernel(npad, e, k):
    @functools.partial(
        pl.kernel,
        mesh=_mesh(),
        out_type=jax.ShapeDtypeStruct((NC, npad), jnp.float32),
        scratch_types=[
            pltpu.VMEM((8, 2, CH), jnp.int32),
            pltpu.VMEM((CH,), jnp.float32),
            pltpu.VMEM_SHARED((npad,), jnp.float32),
            [pltpu.SemaphoreType.DMA] * 8,
            pltpu.SemaphoreType.DMA,
            pltpu.SemaphoreType.DMA,
        ],
    )
    def hist(ei_hbm, dum_hbm, zer_hbm, ones_hbm, out_hbm, idx_v, ones_v,
             hist_sh, isems, ssem, zsem):
        c = lax.axis_index("c")
        s = lax.axis_index("s")
        wid = s * NC + c
        kfull = e // (NW * CH)
        nt = (e // CH) % NW
        kw = kfull + jnp.where(wid < nt, 1, 0)
        e0 = NW * kfull * CH
        base = wid * (kfull * CH)
        rows = npad // NS

        pltpu.async_copy(zer_hbm.at[pl.ds(s * rows, rows)],
                         hist_sh.at[pl.ds(s * rows, rows)], zsem)
        pltpu.sync_copy(ones_hbm, ones_v)

        def idx_load_fast(j, slot):
            pltpu.async_copy(ei_hbm.at[pl.ds(0, 2), pl.ds(base + j * CH, CH)],
                             idx_v.at[slot], isems[slot])

        def idx_load_tail(j, slot):
            off = jnp.where(j < kfull, base + j * CH, e0 + wid * CH)

            @pl.when(j < kw)
            def _():
                pltpu.async_copy(ei_hbm.at[pl.ds(0, 2), pl.ds(off, CH)],
                                 idx_v.at[slot], isems[slot])

            @pl.when(j >= kw)
            def _():
                pltpu.async_copy(dum_hbm, idx_v.at[slot], isems[slot])

        def idx_wait(slot):
            pltpu.make_async_copy(dum_hbm, idx_v.at[slot], isems[slot]).wait()

        for i in range(8):
            idx_load_fast(i, i)
        pltpu.make_async_copy(zer_hbm.at[pl.ds(s * rows, rows)],
                              hist_sh.at[pl.ds(s * rows, rows)], zsem).wait()
        plsc.subcore_barrier()

        def load_static(c, slot):
            if c >= k:
                return
            if c < kfull:
                idx_load_fast(c, slot)
            else:
                idx_load_tail(c, slot)

        def step(m, load):
            for i in range(8):
                idx_wait(i)
                pltpu.async_copy(ones_v, hist_sh.at[idx_v.at[i, 1]], ssem,
                                 add=True)
            for i in range(8):
                pltpu.make_async_copy(ones_v, hist_sh.at[idx_v.at[i, 1]],
                                      ssem).wait()
                if load is not None:
                    load(8 * m + i + 8, i)

        nm = k // 8
        lax.fori_loop(
            0, nm - 2,
            lambda m, cr: (step(m, idx_load_fast), cr)[1], 0, unroll=False)
        step(nm - 2, load_static)
        step(nm - 1, None)
        plsc.subcore_barrier()
        pltpu.sync_copy(hist_sh.at[pl.ds(s * rows, rows)],
                        out_hbm.at[c, pl.ds(s * rows, rows)])

    return hist


def _scatter_kernel(n, npad, d, e, k):
    assert k % 2 == 0

    assert k % 4 == 0
    nm = k // 4

    @functools.partial(
        pl.kernel,
        mesh=_mesh(),
        out_type=jax.ShapeDtypeStruct((NC, npad, d), jnp.float32),
        scratch_types=[
            pltpu.VMEM((4, 2, CH), jnp.int32),
            pltpu.VMEM((2, CH, d), jnp.float32),
            pltpu.VMEM_SHARED((npad, d), jnp.float32),
            [pltpu.SemaphoreType.DMA] * 4,
            [pltpu.SemaphoreType.DMA] * 2,
            [pltpu.SemaphoreType.DMA] * 2,
            pltpu.SemaphoreType.DMA,
        ],
    )
    def scat(y_hbm, ei_hbm, dum_hbm, zer_hbm, out_hbm,
             idx_v, buf_v, acc_sh, isems, gsems, ssems, zsem):
        c = lax.axis_index("c")
        s = lax.axis_index("s")
        wid = s * NC + c
        kfull = e // (NW * CH)
        nt = (e // CH) % NW
        kw = kfull + jnp.where(wid < nt, 1, 0)
        e0 = NW * kfull * CH
        base = wid * (kfull * CH)
        rows = npad // NS

        pltpu.async_copy(zer_hbm.at[pl.ds(s * rows, rows)],
                         acc_sh.at[pl.ds(s * rows, rows)], zsem)

        def idx_load_fast(j, slot):
            pltpu.async_copy(ei_hbm.at[pl.ds(0, 2), pl.ds(base + j * CH, CH)],
                             idx_v.at[slot], isems[slot])

        def idx_load_tail(j, slot):
            off = jnp.where(j < kfull, base + j * CH, e0 + wid * CH)

            @pl.when(j < kw)
            def _():
                pltpu.async_copy(ei_hbm.at[pl.ds(0, 2), pl.ds(off, CH)],
                                 idx_v.at[slot], isems[slot])

            @pl.when(j >= kw)
            def _():
                pltpu.async_copy(dum_hbm, idx_v.at[slot], isems[slot])

        def idx_wait(slot):
            pltpu.make_async_copy(dum_hbm, idx_v.at[slot], isems[slot]).wait()

        def gat_start(islot, bslot):
            pltpu.async_copy(y_hbm.at[idx_v.at[islot, 0]], buf_v.at[bslot],
                             gsems[bslot])

        def gat_wait(islot, bslot):
            pltpu.make_async_copy(y_hbm.at[idx_v.at[islot, 0]],
                                  buf_v.at[bslot], gsems[bslot]).wait()

        for i in range(4):
            idx_load_fast(i, i)
        idx_wait(0)
        gat_start(0, 0)
        pltpu.make_async_copy(zer_hbm.at[pl.ds(s * rows, rows)],
                              acc_sh.at[pl.ds(s * rows, rows)], zsem).wait()
        plsc.subcore_barrier()

        def step(m, load, last=False):
            for i in range(4):
                if not (last and i == 3):
                    idx_wait((i + 1) % 4)
                    gat_start((i + 1) % 4, (i + 1) % 2)
                gat_wait(i % 4, i % 2)
                pltpu.sync_copy(buf_v.at[i % 2], acc_sh.at[idx_v.at[i, 1]],
                                add=True)
                if load is not None:
                    load(4 * m + i + 4, i)

        nm = k // 4
        lax.fori_loop(
            0, nm - 2,
            lambda m, cr: (step(m, idx_load_fast), cr)[1], 0, unroll=False)
        step(nm - 2, idx_load_tail)
        step(nm - 1, None, last=True)
        plsc.subcore_barrier()
        pltpu.sync_copy(acc_sh.at[pl.ds(s * rows, rows)],
                        out_hbm.at[c, pl.ds(s * rows, rows)])

    return scat


def _dis(h_ref, n):
    h = h_ref[...]
    deg = 1.0 + h[0, :n] + h[1, :n]
    return lax.rsqrt(deg)[:, None]


def _mm_body(x_ref, w_ref, o_ref):
    o_ref[...] = jnp.dot(x_ref[...], w_ref[...],
                         preferred_element_type=jnp.float32)


def _scale_body(n, xw_ref, h_ref, y_ref):
    y_ref[...] = xw_ref[...] * _dis(h_ref, n)


def _tc2_body(n, acc_ref, y1_ref, h_ref, w2_ref, b1_ref, y2_ref):
    dis = _dis(h_ref, n)
    o1 = dis * (acc_ref[0, :n, :] + acc_ref[1, :n, :] + y1_ref[...])
    o1 = o1 + b1_ref[...][None, :]
    hmat = jnp.where(o1 >= 0, o1, NEG_SLOPE * o1)
    y2_ref[...] = jnp.dot(hmat, w2_ref[...],
                          preferred_element_type=jnp.float32) * dis


def _tc3_body(n, acc_ref, y2_ref, h_ref, b2_ref, out_ref):
    dis = _dis(h_ref, n)
    o2 = dis * (acc_ref[0, :n, :] + acc_ref[1, :n, :] + y2_ref[...])
    out_ref[...] = o2 + b2_ref[...][None, :]


def kernel(x, edge_index, W1, b1, W2, b2):
    n, d = x.shape
    e = edge_index.shape[1]
    assert e % CH == 0
    npad = (n // (16 * NS) + 1) * 16 * NS
    ndump = npad - n
    k = -(-e // (NW * CH))
    k += (-k) % 8

    i128 = jnp.arange(CH, dtype=jnp.int32)
    dum = jnp.stack([(i128 * 79) % n, n + i128 % ndump])
    zer = jnp.zeros((npad, d), jnp.float32)
    zerh = jnp.zeros((npad,), jnp.float32)

    onesh = jnp.ones((CH,), jnp.float32)
    histp = _hist_kernel(npad, e, k)(edge_index, dum, zerh, onesh)
    scat = _scatter_kernel(n, npad, d, e, k)

    xw1 = pl.pallas_call(
        _mm_body,
        out_shape=jax.ShapeDtypeStruct((n, d), jnp.float32),
    )(x, W1)
    y1 = pl.pallas_call(
        functools.partial(_scale_body, n),
        out_shape=jax.ShapeDtypeStruct((n, d), jnp.float32),
    )(xw1, histp)

    acc1 = scat(y1, edge_index, dum, zer)

    y2 = pl.pallas_call(
        functools.partial(_tc2_body, n),
        out_shape=jax.ShapeDtypeStruct((n, d), jnp.float32),
    )(acc1, y1, histp, W2, b1)

    acc2 = scat(y2, edge_index, dum, zer)

    out = pl.pallas_call(
        functools.partial(_tc3_body, n),
        out_shape=jax.ShapeDtypeStruct((n, d), jnp.float32),
    )(acc2, y2, histp, b2)

    return out
